# Initial kernel scaffold; baseline (speedup 1.0000x reference)
import jax
import jax.numpy as jnp
from jax.experimental import pallas as pl


def kernel(nodes, edge_index_0, edge_index_1, feat_table, W1, b1, W2, b2):
    raise NotImplementedError("write your pallas kernel here")


# speedup vs baseline: 5.1570x; 5.1570x over previous
import functools

import jax
import jax.numpy as jnp
from jax import lax
from jax.experimental import pallas as pl
from jax.experimental.pallas import tpu as pltpu
from jax.experimental.pallas import tpu_sc as plsc

B = 10000
D = 128
HD = D // 2
E = 160000
NREL = 2
NC = 2
NS = 16
NW = NC * NS
LANES = 16

CH = 125
EDGES_PER_TILE = E // NS
NCH = EDGES_PER_TILE // CH

B_PAD = 10240
SEG_ROWS = B_PAD // NS
ZCH = 128
NZ = SEG_ROWS // ZCH
SELF_PER_TILE = B_PAD // NW
SCH = 80
NSCH = SELF_PER_TILE // SCH

_mesh = plsc.VectorSubcoreMesh(core_axis_name="c", subcore_axis_name="s")


@functools.partial(
    pl.kernel,
    out_type=(
        jax.ShapeDtypeStruct((B_PAD, D), jnp.float32),
        jax.ShapeDtypeStruct((NREL * 2, B_PAD, HD), jnp.float32),
        jax.ShapeDtypeStruct((NREL, B_PAD, LANES), jnp.float32),
    ),
    mesh=_mesh,
    compiler_params=pltpu.CompilerParams(use_tc_tiling_on_sc=False),
    scratch_types=[
        pltpu.VMEM((NCH, CH), jnp.int32),
        pltpu.VMEM((NCH, CH), jnp.int32),
        pltpu.VMEM((NCH, CH), jnp.int32),
        pltpu.VMEM((CH, HD), jnp.float32),
        pltpu.VMEM((CH, LANES), jnp.float32),
        pltpu.VMEM((NSCH, SCH), jnp.int32),
        pltpu.VMEM((SCH, D), jnp.float32),
        pltpu.VMEM((ZCH, HD), jnp.float32),
        pltpu.VMEM((ZCH, LANES), jnp.float32),
        pltpu.VMEM_SHARED((B_PAD, HD), jnp.float32),
        pltpu.VMEM_SHARED((B_PAD, LANES), jnp.float32),
        pltpu.SemaphoreType.DMA,
    ],
)
def _aggregate(feat_hbm, feath_hbm, nodes_hbm, srca_hbm, srcb_hbm, dst_hbm,
               self_out, acc_out, cnt_out,
               srca_v, srcb_v, dst_v, rows_v, ones_v, nidx_v, srows_v,
               zrow_v, zcnt_v, acc_sh, cnt_sh, sem):
    c = lax.axis_index("c")
    s = lax.axis_index("s")
    wid = c * NS + s

    zeros16 = jnp.zeros((LANES,), jnp.float32)
    ones16 = jnp.ones((LANES,), jnp.float32)

    def _init_row(i, carry):
        for j in range(HD // LANES):
            zrow_v[i, pl.ds(j * LANES, LANES)] = zeros16
        zcnt_v[i, :] = zeros16
        return carry

    lax.fori_loop(0, ZCH, _init_row, 0)

    def _init_ones(i, carry):
        ones_v[i, :] = ones16
        return carry

    lax.fori_loop(0, CH, _init_ones, 0)

    for j in range(NZ):
        base = s * SEG_ROWS + j * ZCH
        pltpu.sync_copy(zrow_v, acc_sh.at[pl.ds(base, ZCH)])
        pltpu.sync_copy(zcnt_v, cnt_sh.at[pl.ds(base, ZCH)])
    plsc.subcore_barrier()

    pltpu.sync_copy(srca_hbm.at[pl.ds(wid * NCH, NCH)], srca_v)
    pltpu.sync_copy(srcb_hbm.at[pl.ds(wid * NCH, NCH)], srcb_v)
    pltpu.sync_copy(dst_hbm.at[pl.ds(wid * NCH, NCH)], dst_v)

    def _edge_step0(j, carry):
        pltpu.async_copy(feath_hbm.at[srca_v.at[j]], rows_v, sem).wait()
        pltpu.sync_copy(rows_v, acc_sh.at[dst_v.at[j]], add=True)
        pltpu.sync_copy(ones_v, cnt_sh.at[dst_v.at[j]], add=True)
        return carry

    lax.fori_loop(0, NCH, _edge_step0, 0)

    plsc.subcore_barrier()
    for j in range(NZ):
        base = s * SEG_ROWS + j * ZCH
        pltpu.sync_copy(acc_sh.at[pl.ds(base, ZCH)],
                        acc_out.at[2 * c, pl.ds(base, ZCH)])
        pltpu.sync_copy(cnt_sh.at[pl.ds(base, ZCH)],
                        cnt_out.at[c, pl.ds(base, ZCH)])
    for j in range(NZ):
        base = s * SEG_ROWS + j * ZCH
        pltpu.sync_copy(zrow_v, acc_sh.at[pl.ds(base, ZCH)])
    plsc.subcore_barrier()

    def _edge_step1(j, carry):
        pltpu.async_copy(feath_hbm.at[srcb_v.at[j]], rows_v, sem).wait()
        pltpu.sync_copy(rows_v, acc_sh.at[dst_v.at[j]], add=True)
        return carry

    lax.fori_loop(0, NCH, _edge_step1, 0)

    pltpu.sync_copy(nodes_hbm.at[pl.ds(wid * NSCH, NSCH)], nidx_v)
    for t in range(NSCH):
        pltpu.async_copy(feat_hbm.at[nidx_v.at[t]], srows_v, sem).wait()
        pltpu.sync_copy(
            srows_v, self_out.at[pl.ds(wid * SELF_PER_TILE + t * SCH, SCH)])

    plsc.subcore_barrier()
    for j in range(NZ):
        base = s * SEG_ROWS + j * ZCH
        pltpu.sync_copy(acc_sh.at[pl.ds(base, ZCH)],
                        acc_out.at[2 * c + 1, pl.ds(base, ZCH)])


BLK = 2000


def _mlp_body(self_ref, a0l_ref, a0r_ref, a1l_ref, a1r_ref, c0_ref, c1_ref,
              w1a_ref, w1bl_ref, w1br_ref, w1cl_ref, w1cr_ref,
              b1_ref, w2_ref, b2_ref, out_ref):
    inv0 = 1.0 / jnp.maximum(c0_ref[:, 0:1], 1.0)
    inv1 = 1.0 / jnp.maximum(c1_ref[:, 0:1], 1.0)
    h = jnp.dot(self_ref[:], w1a_ref[:], preferred_element_type=jnp.float32)
    h = h + jnp.dot(a0l_ref[:] * inv0, w1bl_ref[:],
                    preferred_element_type=jnp.float32)
    h = h + jnp.dot(a0r_ref[:] * inv0, w1br_ref[:],
                    preferred_element_type=jnp.float32)
    h = h + jnp.dot(a1l_ref[:] * inv1, w1cl_ref[:],
                    preferred_element_type=jnp.float32)
    h = h + jnp.dot(a1r_ref[:] * inv1, w1cr_ref[:],
                    preferred_element_type=jnp.float32)
    h = jnp.tanh(h + b1_ref[:])
    out_ref[:] = jnp.dot(h, w2_ref[:],
                         preferred_element_type=jnp.float32) + b2_ref[:]


def _mlp(self_feats, a0l, a0r, a1l, a1r, c0, c1,
         w1a, w1bl, w1br, w1cl, w1cr, b1, w2, b2):
    row_spec = pl.BlockSpec((BLK, D), lambda i: (i, 0))
    half_spec = pl.BlockSpec((BLK, HD), lambda i: (i, 0))
    cnt_spec = pl.BlockSpec((BLK, LANES), lambda i: (i, 0))

    def full(shape):
        return pl.BlockSpec(shape, lambda i: (0, 0))

    return pl.pallas_call(
        _mlp_body,
        grid=(B // BLK,),
        in_specs=[row_spec, half_spec, half_spec, half_spec, half_spec,
                  cnt_spec, cnt_spec,
                  full((D, D)), full((HD, D)), full((HD, D)),
                  full((HD, D)), full((HD, D)),
                  full((1, D)), full((D, D)), full((1, D))],
        out_specs=row_spec,
        out_shape=jax.ShapeDtypeStruct((B, D), jnp.float32),
    )(self_feats, a0l, a0r, a1l, a1r, c0, c1,
      w1a, w1bl, w1br, w1cl, w1cr, b1, w2, b2)


def kernel(nodes, edge_index_0, edge_index_1, feat_table, W1, b1, W2, b2):
    nodes = nodes.astype(jnp.int32)
    nodes_pad = jnp.pad(nodes, (0, B_PAD - B)).reshape(NW * NSCH, SCH)
    src = jnp.concatenate(
        [edge_index_0[1], edge_index_1[1]]).astype(jnp.int32).reshape(NW * NCH, CH)
    dst = jnp.concatenate(
        [edge_index_0[0], edge_index_1[0]]).astype(jnp.int32).reshape(NW * NCH, CH)
    n_rows = feat_table.shape[0]
    feat_half = jnp.concatenate([feat_table[:, :HD], feat_table[:, HD:]], axis=0)

    self_rows, acc, cnt = _aggregate(
        feat_table, feat_half, nodes_pad, src, src + n_rows, dst)

    w1t = W1.T
    return _mlp(self_rows[:B],
                acc[0, :B], acc[1, :B], acc[2, :B], acc[3, :B],
                cnt[0, :B], cnt[1, :B],
                w1t[:D], w1t[D:D + HD], w1t[D + HD:2 * D],
                w1t[2 * D:2 * D + HD], w1t[2 * D + HD:],
                b1.reshape(1, D), W2.T, b2.reshape(1, D))


# speedup vs baseline: 7.0411x; 1.3653x over previous
import functools

import jax
import jax.numpy as jnp
from jax import lax
from jax.experimental import pallas as pl
from jax.experimental.pallas import tpu as pltpu
from jax.experimental.pallas import tpu_sc as plsc

B = 10000
D = 128
HD = D // 2
E = 160000
NREL = 2
NC = 2
NS = 16
NW = NC * NS
LANES = 16

CH = 125
EDGES_PER_TILE = E // NS
NCH = EDGES_PER_TILE // CH

B_PAD = 10240
SEG_ROWS = B_PAD // NS
ZCH = 128
NZ = SEG_ROWS // ZCH
SELF_PER_TILE = B_PAD // NW
SCH = 80
NSCH = SELF_PER_TILE // SCH

_mesh = plsc.VectorSubcoreMesh(core_axis_name="c", subcore_axis_name="s")


@functools.partial(
    pl.kernel,
    out_type=(
        jax.ShapeDtypeStruct((B_PAD, D), jnp.float32),
        jax.ShapeDtypeStruct((NREL * 2, B_PAD, HD), jnp.float32),
        jax.ShapeDtypeStruct((NREL, B_PAD, LANES), jnp.float32),
    ),
    mesh=_mesh,
    compiler_params=pltpu.CompilerParams(use_tc_tiling_on_sc=False),
    scratch_types=[
        pltpu.VMEM((NCH, CH), jnp.int32),
        pltpu.VMEM((NCH, CH), jnp.int32),
        pltpu.VMEM((NCH, CH), jnp.int32),
        pltpu.VMEM((CH, HD), jnp.float32),
        pltpu.VMEM((CH, HD), jnp.float32),
        pltpu.VMEM((CH, LANES), jnp.float32),
        pltpu.VMEM((NSCH, SCH), jnp.int32),
        pltpu.VMEM((SCH, D), jnp.float32),
        pltpu.VMEM((ZCH, HD), jnp.float32),
        pltpu.VMEM((ZCH, LANES), jnp.float32),
        pltpu.VMEM_SHARED((B_PAD, HD), jnp.float32),
        pltpu.VMEM_SHARED((B_PAD, LANES), jnp.float32),
        pltpu.SemaphoreType.DMA,
        pltpu.SemaphoreType.DMA,
    ],
)
def _aggregate(feat_hbm, feath_hbm, nodes_hbm, srca_hbm, srcb_hbm, dst_hbm,
               self_out, acc_out, cnt_out,
               srca_v, srcb_v, dst_v, rows_a, rows_b, ones_v, nidx_v, srows_v,
               zrow_v, zcnt_v, acc_sh, cnt_sh, sem_a, sem_b):
    c = lax.axis_index("c")
    s = lax.axis_index("s")
    wid = c * NS + s

    zeros16 = jnp.zeros((LANES,), jnp.float32)
    ones16 = jnp.ones((LANES,), jnp.float32)

    def _init_row(i, carry):
        for j in range(HD // LANES):
            zrow_v[i, pl.ds(j * LANES, LANES)] = zeros16
        zcnt_v[i, :] = zeros16
        return carry

    lax.fori_loop(0, ZCH, _init_row, 0)

    def _init_ones(i, carry):
        ones_v[i, :] = ones16
        return carry

    lax.fori_loop(0, CH, _init_ones, 0)

    for j in range(NZ):
        base = s * SEG_ROWS + j * ZCH
        pltpu.sync_copy(zrow_v, acc_sh.at[pl.ds(base, ZCH)])
        pltpu.sync_copy(zcnt_v, cnt_sh.at[pl.ds(base, ZCH)])
    plsc.subcore_barrier()

    pltpu.sync_copy(srca_hbm.at[pl.ds(wid * NCH, NCH)], srca_v)
    pltpu.sync_copy(srcb_hbm.at[pl.ds(wid * NCH, NCH)], srcb_v)
    pltpu.sync_copy(dst_hbm.at[pl.ds(wid * NCH, NCH)], dst_v)

    def _edge_phase(ids_v, do_cnt):
        def _scatter(buf, j):
            pltpu.sync_copy(buf, acc_sh.at[dst_v.at[j]], add=True)
            if do_cnt:
                pltpu.sync_copy(ones_v, cnt_sh.at[dst_v.at[j]], add=True)

        pltpu.async_copy(feath_hbm.at[ids_v.at[0]], rows_a, sem_a)

        def _pair(i, carry):
            j0 = 2 * i
            j1 = j0 + 1
            pltpu.async_copy(feath_hbm.at[ids_v.at[j1]], rows_b, sem_b)
            pltpu.make_async_copy(
                feath_hbm.at[ids_v.at[j0]], rows_a, sem_a).wait()
            _scatter(rows_a, j0)

            @pl.when(j1 + 1 < NCH)
            def _():
                pltpu.async_copy(
                    feath_hbm.at[ids_v.at[j1 + 1]], rows_a, sem_a)

            pltpu.make_async_copy(
                feath_hbm.at[ids_v.at[j1]], rows_b, sem_b).wait()
            _scatter(rows_b, j1)
            return carry

        lax.fori_loop(0, NCH // 2, _pair, 0)

    _edge_phase(srca_v, True)

    plsc.subcore_barrier()
    for j in range(NZ):
        base = s * SEG_ROWS + j * ZCH
        pltpu.sync_copy(acc_sh.at[pl.ds(base, ZCH)],
                        acc_out.at[2 * c, pl.ds(base, ZCH)])
        pltpu.sync_copy(cnt_sh.at[pl.ds(base, ZCH)],
                        cnt_out.at[c, pl.ds(base, ZCH)])
    for j in range(NZ):
        base = s * SEG_ROWS + j * ZCH
        pltpu.sync_copy(zrow_v, acc_sh.at[pl.ds(base, ZCH)])
    plsc.subcore_barrier()

    _edge_phase(srcb_v, False)

    pltpu.sync_copy(nodes_hbm.at[pl.ds(wid * NSCH, NSCH)], nidx_v)
    for t in range(NSCH):
        pltpu.async_copy(feat_hbm.at[nidx_v.at[t]], srows_v, sem_a).wait()
        pltpu.sync_copy(
            srows_v, self_out.at[pl.ds(wid * SELF_PER_TILE + t * SCH, SCH)])

    plsc.subcore_barrier()
    for j in range(NZ):
        base = s * SEG_ROWS + j * ZCH
        pltpu.sync_copy(acc_sh.at[pl.ds(base, ZCH)],
                        acc_out.at[2 * c + 1, pl.ds(base, ZCH)])


BLK = 2000


def _mlp_body(self_ref, a0l_ref, a0r_ref, a1l_ref, a1r_ref, c0_ref, c1_ref,
              w1a_ref, w1bl_ref, w1br_ref, w1cl_ref, w1cr_ref,
              b1_ref, w2_ref, b2_ref, out_ref):
    inv0 = 1.0 / jnp.maximum(c0_ref[:, 0:1], 1.0)
    inv1 = 1.0 / jnp.maximum(c1_ref[:, 0:1], 1.0)
    h = jnp.dot(self_ref[:], w1a_ref[:], preferred_element_type=jnp.float32)
    h = h + jnp.dot(a0l_ref[:] * inv0, w1bl_ref[:],
                    preferred_element_type=jnp.float32)
    h = h + jnp.dot(a0r_ref[:] * inv0, w1br_ref[:],
                    preferred_element_type=jnp.float32)
    h = h + jnp.dot(a1l_ref[:] * inv1, w1cl_ref[:],
                    preferred_element_type=jnp.float32)
    h = h + jnp.dot(a1r_ref[:] * inv1, w1cr_ref[:],
                    preferred_element_type=jnp.float32)
    h = jnp.tanh(h + b1_ref[:])
    out_ref[:] = jnp.dot(h, w2_ref[:],
                         preferred_element_type=jnp.float32) + b2_ref[:]


def _mlp(self_feats, a0l, a0r, a1l, a1r, c0, c1,
         w1a, w1bl, w1br, w1cl, w1cr, b1, w2, b2):
    row_spec = pl.BlockSpec((BLK, D), lambda i: (i, 0))
    half_spec = pl.BlockSpec((BLK, HD), lambda i: (i, 0))
    cnt_spec = pl.BlockSpec((BLK, LANES), lambda i: (i, 0))

    def full(shape):
        return pl.BlockSpec(shape, lambda i: (0, 0))

    return pl.pallas_call(
        _mlp_body,
        grid=(B // BLK,),
        in_specs=[row_spec, half_spec, half_spec, half_spec, half_spec,
                  cnt_spec, cnt_spec,
                  full((D, D)), full((HD, D)), full((HD, D)),
                  full((HD, D)), full((HD, D)),
                  full((1, D)), full((D, D)), full((1, D))],
        out_specs=row_spec,
        out_shape=jax.ShapeDtypeStruct((B, D), jnp.float32),
    )(self_feats, a0l, a0r, a1l, a1r, c0, c1,
      w1a, w1bl, w1br, w1cl, w1cr, b1, w2, b2)


def kernel(nodes, edge_index_0, edge_index_1, feat_table, W1, b1, W2, b2):
    nodes = nodes.astype(jnp.int32)
    nodes_pad = jnp.pad(nodes, (0, B_PAD - B)).reshape(NW * NSCH, SCH)
    src = jnp.concatenate(
        [edge_index_0[1], edge_index_1[1]]).astype(jnp.int32).reshape(NW * NCH, CH)
    dst = jnp.concatenate(
        [edge_index_0[0], edge_index_1[0]]).astype(jnp.int32).reshape(NW * NCH, CH)
    n_rows = feat_table.shape[0]
    feat_half = jnp.concatenate([feat_table[:, :HD], feat_table[:, HD:]], axis=0)

    self_rows, acc, cnt = _aggregate(
        feat_table, feat_half, nodes_pad, src, src + n_rows, dst)

    w1t = W1.T
    return _mlp(self_rows[:B],
                acc[0, :B], acc[1, :B], acc[2, :B], acc[3, :B],
                cnt[0, :B], cnt[1, :B],
                w1t[:D], w1t[D:D + HD], w1t[D + HD:2 * D],
                w1t[2 * D:2 * D + HD], w1t[2 * D + HD:],
                b1.reshape(1, D), W2.T, b2.reshape(1, D))


# speedup vs baseline: 7.1510x; 1.0156x over previous
import functools

import jax
import jax.numpy as jnp
from jax import lax
from jax.experimental import pallas as pl
from jax.experimental.pallas import tpu as pltpu
from jax.experimental.pallas import tpu_sc as plsc

B = 10000
D = 128
HD = D // 2
E = 160000
NREL = 2
NC = 2
NS = 16
NW = NC * NS
LANES = 16

CH = 125
EDGES_PER_TILE = E // NS
NCH = EDGES_PER_TILE // CH

B_PAD = 10240
SEG_ROWS = B_PAD // NS
ZCH = 128
NZ = SEG_ROWS // ZCH
SELF_PER_TILE = B_PAD // NW
SCH = 80
NSCH = SELF_PER_TILE // SCH

_mesh = plsc.VectorSubcoreMesh(core_axis_name="c", subcore_axis_name="s")


@functools.partial(
    pl.kernel,
    out_type=(
        jax.ShapeDtypeStruct((B_PAD, D), jnp.float32),
        jax.ShapeDtypeStruct((NREL * 2, B_PAD, HD), jnp.float32),
        jax.ShapeDtypeStruct((NREL, B_PAD, LANES), jnp.float32),
    ),
    mesh=_mesh,
    compiler_params=pltpu.CompilerParams(use_tc_tiling_on_sc=False),
    scratch_types=[
        pltpu.VMEM((NCH, CH), jnp.int32),
        pltpu.VMEM((NCH, CH), jnp.int32),
        pltpu.VMEM((NCH, CH), jnp.int32),
        pltpu.VMEM((CH, HD), jnp.float32),
        pltpu.VMEM((CH, HD), jnp.float32),
        pltpu.VMEM((CH, LANES), jnp.float32),
        pltpu.VMEM((NSCH, SCH), jnp.int32),
        pltpu.VMEM((SCH, D), jnp.float32),
        pltpu.VMEM((ZCH, HD), jnp.float32),
        pltpu.VMEM((ZCH, LANES), jnp.float32),
        pltpu.VMEM_SHARED((B_PAD, HD), jnp.float32),
        pltpu.VMEM_SHARED((B_PAD, LANES), jnp.float32),
        pltpu.SemaphoreType.DMA,
        pltpu.SemaphoreType.DMA,
        pltpu.SemaphoreType.DMA,
    ],
)
def _aggregate(feat_hbm, feath_hbm, nodes_hbm, srca_hbm, srcb_hbm, dst_hbm,
               self_out, acc_out, cnt_out,
               srca_v, srcb_v, dst_v, rows_a, rows_b, ones_v, nidx_v, srows_v,
               zrow_v, zcnt_v, acc_sh, cnt_sh, sem_a, sem_b, sem_c):
    c = lax.axis_index("c")
    s = lax.axis_index("s")
    wid = c * NS + s

    zeros16 = jnp.zeros((LANES,), jnp.float32)
    ones16 = jnp.ones((LANES,), jnp.float32)

    def _init_row(i, carry):
        for j in range(HD // LANES):
            zrow_v[i, pl.ds(j * LANES, LANES)] = zeros16
        zcnt_v[i, :] = zeros16
        return carry

    lax.fori_loop(0, ZCH, _init_row, 0)

    def _init_ones(i, carry):
        ones_v[i, :] = ones16
        return carry

    lax.fori_loop(0, CH, _init_ones, 0)

    for j in range(NZ):
        base = s * SEG_ROWS + j * ZCH
        pltpu.sync_copy(zrow_v, acc_sh.at[pl.ds(base, ZCH)])
        pltpu.sync_copy(zcnt_v, cnt_sh.at[pl.ds(base, ZCH)])
    plsc.subcore_barrier()

    pltpu.sync_copy(srca_hbm.at[pl.ds(wid * NCH, NCH)], srca_v)
    pltpu.sync_copy(srcb_hbm.at[pl.ds(wid * NCH, NCH)], srcb_v)
    pltpu.sync_copy(dst_hbm.at[pl.ds(wid * NCH, NCH)], dst_v)

    def _edge_phase(ids_v, do_cnt):
        def _gather_src(j):
            return feath_hbm.at[ids_v.at[j]]

        def _scatter(buf, j):
            pltpu.sync_copy(buf, acc_sh.at[dst_v.at[j]], add=True)
            if do_cnt:
                pltpu.async_copy(
                    ones_v, cnt_sh.at[dst_v.at[j]], sem_c, add=True)

        pltpu.async_copy(_gather_src(0), rows_a, sem_a)

        def _pair(i, carry):
            j0 = 2 * i
            j1 = j0 + 1
            pltpu.async_copy(_gather_src(j1), rows_b, sem_b)
            pltpu.make_async_copy(_gather_src(j0), rows_a, sem_a).wait()
            _scatter(rows_a, j0)

            @pl.when(j1 + 1 < NCH)
            def _():
                pltpu.async_copy(_gather_src(j1 + 1), rows_a, sem_a)

            pltpu.make_async_copy(_gather_src(j1), rows_b, sem_b).wait()
            _scatter(rows_b, j1)
            return carry

        lax.fori_loop(0, NCH // 2, _pair, 0)

        if do_cnt:
            def _drain(j, carry):
                pltpu.make_async_copy(
                    ones_v, cnt_sh.at[dst_v.at[j]], sem_c).wait()
                return carry

            lax.fori_loop(0, NCH, _drain, 0)

    _edge_phase(srca_v, True)

    plsc.subcore_barrier()
    for j in range(NZ):
        base = s * SEG_ROWS + j * ZCH
        pltpu.sync_copy(acc_sh.at[pl.ds(base, ZCH)],
                        acc_out.at[2 * c, pl.ds(base, ZCH)])
        pltpu.sync_copy(cnt_sh.at[pl.ds(base, ZCH)],
                        cnt_out.at[c, pl.ds(base, ZCH)])
    for j in range(NZ):
        base = s * SEG_ROWS + j * ZCH
        pltpu.sync_copy(zrow_v, acc_sh.at[pl.ds(base, ZCH)])
    plsc.subcore_barrier()

    _edge_phase(srcb_v, False)

    pltpu.sync_copy(nodes_hbm.at[pl.ds(wid * NSCH, NSCH)], nidx_v)
    for t in range(NSCH):
        pltpu.async_copy(feat_hbm.at[nidx_v.at[t]], srows_v, sem_a).wait()
        pltpu.sync_copy(
            srows_v, self_out.at[pl.ds(wid * SELF_PER_TILE + t * SCH, SCH)])

    plsc.subcore_barrier()
    for j in range(NZ):
        base = s * SEG_ROWS + j * ZCH
        pltpu.sync_copy(acc_sh.at[pl.ds(base, ZCH)],
                        acc_out.at[2 * c + 1, pl.ds(base, ZCH)])


BLK = 2000


def _mlp_body(self_ref, a0l_ref, a0r_ref, a1l_ref, a1r_ref, c0_ref, c1_ref,
              w1a_ref, w1bl_ref, w1br_ref, w1cl_ref, w1cr_ref,
              b1_ref, w2_ref, b2_ref, out_ref):
    inv0 = 1.0 / jnp.maximum(c0_ref[:, 0:1], 1.0)
    inv1 = 1.0 / jnp.maximum(c1_ref[:, 0:1], 1.0)
    h = jnp.dot(self_ref[:], w1a_ref[:], preferred_element_type=jnp.float32)
    h = h + jnp.dot(a0l_ref[:] * inv0, w1bl_ref[:],
                    preferred_element_type=jnp.float32)
    h = h + jnp.dot(a0r_ref[:] * inv0, w1br_ref[:],
                    preferred_element_type=jnp.float32)
    h = h + jnp.dot(a1l_ref[:] * inv1, w1cl_ref[:],
                    preferred_element_type=jnp.float32)
    h = h + jnp.dot(a1r_ref[:] * inv1, w1cr_ref[:],
                    preferred_element_type=jnp.float32)
    h = jnp.tanh(h + b1_ref[:])
    out_ref[:] = jnp.dot(h, w2_ref[:],
                         preferred_element_type=jnp.float32) + b2_ref[:]


def _mlp(self_feats, a0l, a0r, a1l, a1r, c0, c1,
         w1a, w1bl, w1br, w1cl, w1cr, b1, w2, b2):
    row_spec = pl.BlockSpec((BLK, D), lambda i: (i, 0))
    half_spec = pl.BlockSpec((BLK, HD), lambda i: (i, 0))
    cnt_spec = pl.BlockSpec((BLK, LANES), lambda i: (i, 0))

    def full(shape):
        return pl.BlockSpec(shape, lambda i: (0, 0))

    return pl.pallas_call(
        _mlp_body,
        grid=(B // BLK,),
        in_specs=[row_spec, half_spec, half_spec, half_spec, half_spec,
                  cnt_spec, cnt_spec,
                  full((D, D)), full((HD, D)), full((HD, D)),
                  full((HD, D)), full((HD, D)),
                  full((1, D)), full((D, D)), full((1, D))],
        out_specs=row_spec,
        out_shape=jax.ShapeDtypeStruct((B, D), jnp.float32),
    )(self_feats, a0l, a0r, a1l, a1r, c0, c1,
      w1a, w1bl, w1br, w1cl, w1cr, b1, w2, b2)


def kernel(nodes, edge_index_0, edge_index_1, feat_table, W1, b1, W2, b2):
    nodes = nodes.astype(jnp.int32)
    nodes_pad = jnp.pad(nodes, (0, B_PAD - B)).reshape(NW * NSCH, SCH)
    src = jnp.concatenate(
        [edge_index_0[1], edge_index_1[1]]).astype(jnp.int32).reshape(NW * NCH, CH)
    dst = jnp.concatenate(
        [edge_index_0[0], edge_index_1[0]]).astype(jnp.int32).reshape(NW * NCH, CH)
    n_rows = feat_table.shape[0]
    feat_half = jnp.concatenate([feat_table[:, :HD], feat_table[:, HD:]], axis=0)

    self_rows, acc, cnt = _aggregate(
        feat_table, feat_half, nodes_pad, src, src + n_rows, dst)

    w1t = W1.T
    return _mlp(self_rows[:B],
                acc[0, :B], acc[1, :B], acc[2, :B], acc[3, :B],
                cnt[0, :B], cnt[1, :B],
                w1t[:D], w1t[D:D + HD], w1t[D + HD:2 * D],
                w1t[2 * D:2 * D + HD], w1t[2 * D + HD:],
                b1.reshape(1, D), W2.T, b2.reshape(1, D))


# speedup vs baseline: 7.1642x; 1.0018x over previous
import functools

import jax
import jax.numpy as jnp
from jax import lax
from jax.experimental import pallas as pl
from jax.experimental.pallas import tpu as pltpu
from jax.experimental.pallas import tpu_sc as plsc

B = 10000
D = 128
HD = D // 2
E = 160000
NREL = 2
NC = 2
NS = 16
NW = NC * NS
LANES = 16

CH = 125
EDGES_PER_TILE = E // NS
NCH = EDGES_PER_TILE // CH

B_PAD = 10240
SEG_ROWS = B_PAD // NS
ZCH = 128
NZ = SEG_ROWS // ZCH
SELF_PER_TILE = B_PAD // NW
SCH = 80
NSCH = SELF_PER_TILE // SCH

_mesh = plsc.VectorSubcoreMesh(core_axis_name="c", subcore_axis_name="s")


@functools.partial(
    pl.kernel,
    out_type=(
        jax.ShapeDtypeStruct((B_PAD, D), jnp.float32),
        jax.ShapeDtypeStruct((NREL * 2, B_PAD, HD), jnp.float32),
        jax.ShapeDtypeStruct((NREL, B_PAD, LANES), jnp.float32),
    ),
    mesh=_mesh,
    compiler_params=pltpu.CompilerParams(use_tc_tiling_on_sc=False),
    scratch_types=[
        pltpu.VMEM((NCH, CH), jnp.int32),
        pltpu.VMEM((NCH, CH), jnp.int32),
        pltpu.VMEM((NCH, CH), jnp.int32),
        pltpu.VMEM((CH, HD), jnp.float32),
        pltpu.VMEM((CH, HD), jnp.float32),
        pltpu.VMEM((CH, LANES), jnp.float32),
        pltpu.VMEM((NSCH, SCH), jnp.int32),
        pltpu.VMEM((SCH, D), jnp.float32),
        pltpu.VMEM((ZCH, HD), jnp.float32),
        pltpu.VMEM((ZCH, LANES), jnp.float32),
        pltpu.VMEM_SHARED((B_PAD, HD), jnp.float32),
        pltpu.VMEM_SHARED((B_PAD, LANES), jnp.float32),
        pltpu.SemaphoreType.DMA,
        pltpu.SemaphoreType.DMA,
        pltpu.SemaphoreType.DMA,
    ],
)
def _aggregate(feat_hbm, feath_hbm, nodes_hbm, srca_hbm, srcb_hbm, dst_hbm,
               self_out, acc_out, cnt_out,
               srca_v, srcb_v, dst_v, rows_a, rows_b, ones_v, nidx_v, srows_v,
               zrow_v, zcnt_v, acc_sh, cnt_sh, sem_a, sem_b, sem_c):
    c = lax.axis_index("c")
    s = lax.axis_index("s")
    wid = c * NS + s

    zeros16 = jnp.zeros((LANES,), jnp.float32)
    ones16 = jnp.ones((LANES,), jnp.float32)

    def _init_row(i, carry):
        for j in range(HD // LANES):
            zrow_v[i, pl.ds(j * LANES, LANES)] = zeros16
        zcnt_v[i, :] = zeros16
        return carry

    lax.fori_loop(0, ZCH, _init_row, 0)

    def _init_ones(i, carry):
        ones_v[i, :] = ones16
        return carry

    lax.fori_loop(0, CH, _init_ones, 0)

    for j in range(NZ):
        base = s * SEG_ROWS + j * ZCH
        pltpu.sync_copy(zrow_v, acc_sh.at[pl.ds(base, ZCH)])
        pltpu.sync_copy(zcnt_v, cnt_sh.at[pl.ds(base, ZCH)])
    plsc.subcore_barrier()

    pltpu.sync_copy(srca_hbm.at[pl.ds(wid * NCH, NCH)], srca_v)
    pltpu.sync_copy(srcb_hbm.at[pl.ds(wid * NCH, NCH)], srcb_v)
    pltpu.sync_copy(dst_hbm.at[pl.ds(wid * NCH, NCH)], dst_v)

    def _edge_phase(ids_v, do_cnt):
        def _gather_src(j):
            return feath_hbm.at[ids_v.at[j]]

        def _scatter(buf, j):
            pltpu.sync_copy(buf, acc_sh.at[dst_v.at[j]], add=True)
            if do_cnt:
                pltpu.async_copy(
                    ones_v, cnt_sh.at[dst_v.at[j]], sem_c, add=True)

        pltpu.async_copy(_gather_src(0), rows_a, sem_a)

        def _pair(i, carry):
            j0 = 2 * i
            j1 = j0 + 1
            pltpu.async_copy(_gather_src(j1), rows_b, sem_b)
            pltpu.make_async_copy(_gather_src(j0), rows_a, sem_a).wait()
            _scatter(rows_a, j0)

            @pl.when(j1 + 1 < NCH)
            def _():
                pltpu.async_copy(_gather_src(j1 + 1), rows_a, sem_a)

            pltpu.make_async_copy(_gather_src(j1), rows_b, sem_b).wait()
            _scatter(rows_b, j1)
            return carry

        lax.fori_loop(0, NCH // 2, _pair, 0)

        if do_cnt:
            def _drain(j, carry):
                pltpu.make_async_copy(
                    ones_v, cnt_sh.at[dst_v.at[j]], sem_c).wait()
                return carry

            lax.fori_loop(0, NCH, _drain, 0)

    _edge_phase(srca_v, True)

    plsc.subcore_barrier()
    for j in range(NZ):
        base = s * SEG_ROWS + j * ZCH
        pltpu.sync_copy(acc_sh.at[pl.ds(base, ZCH)],
                        acc_out.at[2 * c, pl.ds(base, ZCH)])
        pltpu.sync_copy(cnt_sh.at[pl.ds(base, ZCH)],
                        cnt_out.at[c, pl.ds(base, ZCH)])
    for j in range(NZ):
        base = s * SEG_ROWS + j * ZCH
        pltpu.sync_copy(zrow_v, acc_sh.at[pl.ds(base, ZCH)])
    plsc.subcore_barrier()

    _edge_phase(srcb_v, False)

    pltpu.sync_copy(nodes_hbm.at[pl.ds(wid * NSCH, NSCH)], nidx_v)
    for t in range(NSCH):
        pltpu.async_copy(feat_hbm.at[nidx_v.at[t]], srows_v, sem_a).wait()
        pltpu.sync_copy(
            srows_v, self_out.at[pl.ds(wid * SELF_PER_TILE + t * SCH, SCH)])

    plsc.subcore_barrier()
    for j in range(NZ):
        base = s * SEG_ROWS + j * ZCH
        pltpu.sync_copy(acc_sh.at[pl.ds(base, ZCH)],
                        acc_out.at[2 * c + 1, pl.ds(base, ZCH)])


BLK = 2000


def _tdot(x, w):
    return lax.dot_general(x, w, (((1,), (1,)), ((), ())),
                           preferred_element_type=jnp.float32)


def _mlp_body(self_ref, a0l_ref, a0r_ref, a1l_ref, a1r_ref, c0_ref, c1_ref,
              w1_ref, b1_ref, w2_ref, b2_ref, out_ref):
    inv0 = 1.0 / jnp.maximum(c0_ref[:, 0:1], 1.0)
    inv1 = 1.0 / jnp.maximum(c1_ref[:, 0:1], 1.0)
    n0 = jnp.concatenate([a0l_ref[:], a0r_ref[:]], axis=1) * inv0
    n1 = jnp.concatenate([a1l_ref[:], a1r_ref[:]], axis=1) * inv1
    h = _tdot(self_ref[:], w1_ref[:, :D])
    h = h + _tdot(n0, w1_ref[:, D:2 * D])
    h = h + _tdot(n1, w1_ref[:, 2 * D:])
    h = jnp.tanh(h + b1_ref[:])
    out_ref[:] = _tdot(h, w2_ref[:]) + b2_ref[:]


def _mlp(self_feats, a0l, a0r, a1l, a1r, c0, c1, w1, b1, w2, b2):
    row_spec = pl.BlockSpec((BLK, D), lambda i: (i, 0))
    half_spec = pl.BlockSpec((BLK, HD), lambda i: (i, 0))
    cnt_spec = pl.BlockSpec((BLK, LANES), lambda i: (i, 0))

    def full(shape):
        return pl.BlockSpec(shape, lambda *_: (0,) * len(shape))

    return pl.pallas_call(
        _mlp_body,
        grid=(B // BLK,),
        in_specs=[row_spec, half_spec, half_spec, half_spec, half_spec,
                  cnt_spec, cnt_spec,
                  full((D, 3 * D)), full((D,)), full((D, D)), full((D,))],
        out_specs=row_spec,
        out_shape=jax.ShapeDtypeStruct((B, D), jnp.float32),
    )(self_feats, a0l, a0r, a1l, a1r, c0, c1, w1, b1, w2, b2)


def kernel(nodes, edge_index_0, edge_index_1, feat_table, W1, b1, W2, b2):
    nodes = nodes.astype(jnp.int32)
    nodes_pad = jnp.pad(nodes, (0, B_PAD - B)).reshape(NW * NSCH, SCH)
    src = jnp.concatenate(
        [edge_index_0[1], edge_index_1[1]]).astype(jnp.int32).reshape(NW * NCH, CH)
    dst = jnp.concatenate(
        [edge_index_0[0], edge_index_1[0]]).astype(jnp.int32).reshape(NW * NCH, CH)
    n_rows = feat_table.shape[0]
    feat_half = jnp.concatenate([feat_table[:, :HD], feat_table[:, HD:]], axis=0)

    self_rows, acc, cnt = _aggregate(
        feat_table, feat_half, nodes_pad, src, src + n_rows, dst)

    return _mlp(self_rows, acc[0], acc[1], acc[2], acc[3],
                cnt[0], cnt[1], W1, b1, W2, b2)


# speedup vs baseline: 7.5561x; 1.0547x over previous
import functools

import jax
import jax.numpy as jnp
from jax import lax
from jax.experimental import pallas as pl
from jax.experimental.pallas import tpu as pltpu
from jax.experimental.pallas import tpu_sc as plsc

B = 10000
D = 128
HD = D // 2
E = 160000
NREL = 2
NC = 2
NS = 16
NW = NC * NS
LANES = 16

CH = 125
EDGES_PER_TILE = E // NS
NCH = EDGES_PER_TILE // CH

B_PAD = 10240
SEG_ROWS = B_PAD // NS
ZCH = 128
NZ = SEG_ROWS // ZCH
SELF_PER_TILE = B_PAD // NW
SCH = 80
NSCH = SELF_PER_TILE // SCH

_mesh = plsc.VectorSubcoreMesh(core_axis_name="c", subcore_axis_name="s")


@functools.partial(
    pl.kernel,
    out_type=(
        jax.ShapeDtypeStruct((B_PAD, D), jnp.float32),
        jax.ShapeDtypeStruct((NREL * 2, B_PAD, HD), jnp.float32),
        jax.ShapeDtypeStruct((NREL, B_PAD, LANES), jnp.float32),
    ),
    mesh=_mesh,
    compiler_params=pltpu.CompilerParams(use_tc_tiling_on_sc=False),
    scratch_types=[
        pltpu.VMEM((NCH, CH), jnp.int32),
        pltpu.VMEM((NCH, CH), jnp.int32),
        pltpu.VMEM((ZCH, HD), jnp.float32),
        pltpu.VMEM((ZCH, HD), jnp.float32),
        pltpu.VMEM((ZCH, HD), jnp.float32),
        pltpu.VMEM((ZCH, HD), jnp.float32),
        pltpu.VMEM((CH, LANES), jnp.float32),
        pltpu.VMEM((NSCH, SCH), jnp.int32),
        pltpu.VMEM((SCH, D), jnp.float32),
        pltpu.VMEM((ZCH, LANES), jnp.float32),
        pltpu.VMEM_SHARED((B_PAD, HD), jnp.float32),
        pltpu.VMEM_SHARED((B_PAD, LANES), jnp.float32),
        pltpu.SemaphoreType.DMA,
        pltpu.SemaphoreType.DMA,
        pltpu.SemaphoreType.DMA,
    ],
)
def _aggregate(feat_hbm, feath_hbm, nodes_hbm, srca_hbm, srcb_hbm, dst_hbm,
               self_out, acc_out, cnt_out,
               src_v, dst_v, rows_a, rows_b, rows_c, rows_d,
               ones_v, nidx_v, srows_v,
               zcnt_v, acc_sh, cnt_sh, sem_g, sem_s, sem_c):
    c = lax.axis_index("c")
    s = lax.axis_index("s")
    wid = c * NS + s

    zeros16 = jnp.zeros((LANES,), jnp.float32)
    ones16 = jnp.ones((LANES,), jnp.float32)

    def _zero_rows_a():
        def _zr(i, carry):
            for j in range(HD // LANES):
                rows_a[i, pl.ds(j * LANES, LANES)] = zeros16
            return carry

        lax.fori_loop(0, ZCH, _zr, 0)

    _zero_rows_a()

    def _init_row(i, carry):
        zcnt_v[i, :] = zeros16
        return carry

    lax.fori_loop(0, ZCH, _init_row, 0)

    def _init_ones(i, carry):
        ones_v[i, :] = ones16
        return carry

    lax.fori_loop(0, CH, _init_ones, 0)

    def _zero_acc():
        for j in range(NZ):
            pltpu.sync_copy(
                rows_a, acc_sh.at[pl.ds(s * SEG_ROWS + j * ZCH, ZCH)])

    _zero_acc()
    for j in range(NZ):
        pltpu.sync_copy(zcnt_v, cnt_sh.at[pl.ds(s * SEG_ROWS + j * ZCH, ZCH)])
    plsc.subcore_barrier()

    pltpu.sync_copy(srca_hbm.at[pl.ds(wid * NCH, NCH)], src_v)
    pltpu.sync_copy(dst_hbm.at[pl.ds(wid * NCH, NCH)], dst_v)

    bufs = (rows_a, rows_b, rows_c, rows_d)

    def _edge_phase(do_cnt):
        def _gather_src(j):
            return feath_hbm.at[src_v.at[j]]

        def _rows(buf):
            return buf.at[pl.ds(0, CH)]

        pltpu.async_copy(_gather_src(0), _rows(rows_a), sem_g)
        pltpu.async_copy(_gather_src(1), _rows(rows_b), sem_g)

        def _quad(i, carry):
            for t in range(4):
                j = 4 * i + t
                buf = bufs[t]
                bufm2 = bufs[(t + 2) % 4]
                pltpu.make_async_copy(
                    _gather_src(j), _rows(buf), sem_g).wait()
                pltpu.async_copy(
                    _rows(buf), acc_sh.at[dst_v.at[j]], sem_s, add=True)
                if do_cnt:
                    pltpu.async_copy(
                        ones_v, cnt_sh.at[dst_v.at[j]], sem_c, add=True)

                @pl.when(j >= 2)
                def _():
                    pltpu.make_async_copy(
                        _rows(bufm2), acc_sh.at[dst_v.at[j - 2]], sem_s).wait()

                @pl.when(j + 2 < NCH)
                def _():
                    pltpu.async_copy(_gather_src(j + 2), _rows(bufm2), sem_g)
            return carry

        lax.fori_loop(0, NCH // 4, _quad, 0)

        pltpu.make_async_copy(
            _rows(bufs[2]), acc_sh.at[dst_v.at[NCH - 2]], sem_s).wait()
        pltpu.make_async_copy(
            _rows(bufs[3]), acc_sh.at[dst_v.at[NCH - 1]], sem_s).wait()

        if do_cnt:
            def _drain(j, carry):
                pltpu.make_async_copy(
                    ones_v, cnt_sh.at[dst_v.at[j]], sem_c).wait()
                return carry

            lax.fori_loop(0, NCH, _drain, 0)

    _edge_phase(True)

    plsc.subcore_barrier()
    for j in range(NZ):
        base = s * SEG_ROWS + j * ZCH
        pltpu.sync_copy(acc_sh.at[pl.ds(base, ZCH)],
                        acc_out.at[2 * c, pl.ds(base, ZCH)])
        pltpu.sync_copy(cnt_sh.at[pl.ds(base, ZCH)],
                        cnt_out.at[c, pl.ds(base, ZCH)])
    _zero_rows_a()
    _zero_acc()
    pltpu.sync_copy(srcb_hbm.at[pl.ds(wid * NCH, NCH)], src_v)
    plsc.subcore_barrier()

    _edge_phase(False)

    pltpu.sync_copy(nodes_hbm.at[pl.ds(wid * NSCH, NSCH)], nidx_v)
    for t in range(NSCH):
        pltpu.async_copy(feat_hbm.at[nidx_v.at[t]], srows_v, sem_g).wait()
        pltpu.sync_copy(
            srows_v, self_out.at[pl.ds(wid * SELF_PER_TILE + t * SCH, SCH)])

    plsc.subcore_barrier()
    for j in range(NZ):
        base = s * SEG_ROWS + j * ZCH
        pltpu.sync_copy(acc_sh.at[pl.ds(base, ZCH)],
                        acc_out.at[2 * c + 1, pl.ds(base, ZCH)])


BLK = 2000


def _tdot(x, w):
    return lax.dot_general(x, w, (((1,), (1,)), ((), ())),
                           preferred_element_type=jnp.float32)


def _mlp_body(self_ref, a0l_ref, a0r_ref, a1l_ref, a1r_ref, c0_ref, c1_ref,
              w1_ref, b1_ref, w2_ref, b2_ref, out_ref):
    inv0 = 1.0 / jnp.maximum(c0_ref[:, 0:1], 1.0)
    inv1 = 1.0 / jnp.maximum(c1_ref[:, 0:1], 1.0)
    n0 = jnp.concatenate([a0l_ref[:], a0r_ref[:]], axis=1) * inv0
    n1 = jnp.concatenate([a1l_ref[:], a1r_ref[:]], axis=1) * inv1
    h = _tdot(self_ref[:], w1_ref[:, :D])
    h = h + _tdot(n0, w1_ref[:, D:2 * D])
    h = h + _tdot(n1, w1_ref[:, 2 * D:])
    h = jnp.tanh(h + b1_ref[:])
    out_ref[:] = _tdot(h, w2_ref[:]) + b2_ref[:]


def _mlp(self_feats, a0l, a0r, a1l, a1r, c0, c1, w1, b1, w2, b2):
    row_spec = pl.BlockSpec((BLK, D), lambda i: (i, 0))
    half_spec = pl.BlockSpec((BLK, HD), lambda i: (i, 0))
    cnt_spec = pl.BlockSpec((BLK, LANES), lambda i: (i, 0))

    def full(shape):
        return pl.BlockSpec(shape, lambda *_: (0,) * len(shape))

    return pl.pallas_call(
        _mlp_body,
        grid=(B // BLK,),
        in_specs=[row_spec, half_spec, half_spec, half_spec, half_spec,
                  cnt_spec, cnt_spec,
                  full((D, 3 * D)), full((D,)), full((D, D)), full((D,))],
        out_specs=row_spec,
        out_shape=jax.ShapeDtypeStruct((B, D), jnp.float32),
    )(self_feats, a0l, a0r, a1l, a1r, c0, c1, w1, b1, w2, b2)


def kernel(nodes, edge_index_0, edge_index_1, feat_table, W1, b1, W2, b2):
    nodes = nodes.astype(jnp.int32)
    nodes_pad = jnp.pad(nodes, (0, B_PAD - B)).reshape(NW * NSCH, SCH)
    src = jnp.concatenate(
        [edge_index_0[1], edge_index_1[1]]).astype(jnp.int32).reshape(NW * NCH, CH)
    dst = jnp.concatenate(
        [edge_index_0[0], edge_index_1[0]]).astype(jnp.int32).reshape(NW * NCH, CH)
    n_rows = feat_table.shape[0]
    feat_half = jnp.concatenate([feat_table[:, :HD], feat_table[:, HD:]], axis=0)

    self_rows, acc, cnt = _aggregate(
        feat_table, feat_half, nodes_pad, src, src + n_rows, dst)

    return _mlp(self_rows, acc[0], acc[1], acc[2], acc[3],
                cnt[0], cnt[1], W1, b1, W2, b2)


# speedup vs baseline: 8.6105x; 1.1396x over previous
import functools

import jax
import jax.numpy as jnp
from jax import lax
from jax.experimental import pallas as pl
from jax.experimental.pallas import tpu as pltpu
from jax.experimental.pallas import tpu_sc as plsc

B = 10000
D = 128
HD = D // 2
E = 160000
NREL = 2
NC = 2
NS = 16
NW = NC * NS
LANES = 16

CH = 125
EDGES_PER_TILE = E // NS
NCH = EDGES_PER_TILE // CH

B_PAD = 10240
SEG_ROWS = B_PAD // NS
ZCH = 128
NZ = SEG_ROWS // ZCH
SELF_TILES = 25
SELF_PER_TILE = B // SELF_TILES
SCH = 80
NSCH = SELF_PER_TILE // SCH

_mesh = plsc.VectorSubcoreMesh(core_axis_name="c", subcore_axis_name="s")


@functools.partial(
    pl.kernel,
    out_type=(
        jax.ShapeDtypeStruct((B, D), jnp.float32),
        jax.ShapeDtypeStruct((NREL * 2, B_PAD, HD), jnp.float32),
        jax.ShapeDtypeStruct((NREL, B_PAD, LANES), jnp.float32),
    ),
    mesh=_mesh,
    compiler_params=pltpu.CompilerParams(use_tc_tiling_on_sc=False),
    scratch_types=[
        pltpu.VMEM((NCH, CH), jnp.int32),
        pltpu.VMEM((NCH, CH), jnp.int32),
        pltpu.VMEM((ZCH, HD), jnp.float32),
        pltpu.VMEM((ZCH, HD), jnp.float32),
        pltpu.VMEM((ZCH, HD), jnp.float32),
        pltpu.VMEM((ZCH, HD), jnp.float32),
        pltpu.VMEM((CH, LANES), jnp.float32),
        pltpu.VMEM((NSCH, SCH), jnp.int32),
        pltpu.VMEM((SCH, D), jnp.float32),
        pltpu.VMEM((ZCH, LANES), jnp.float32),
        pltpu.VMEM_SHARED((B_PAD, HD), jnp.float32),
        pltpu.VMEM_SHARED((B_PAD, LANES), jnp.float32),
        pltpu.SemaphoreType.DMA,
        pltpu.SemaphoreType.DMA,
        pltpu.SemaphoreType.DMA,
    ],
)
def _aggregate(feat_hbm, featl_hbm, featr_hbm, nodes_hbm, e0_hbm, e1_hbm,
               self_out, acc_out, cnt_out,
               src_v, dst_v, rows_a, rows_b, rows_c, rows_d,
               ones_v, nidx_v, srows_v,
               zcnt_v, acc_sh, cnt_sh, sem_g, sem_s, sem_c):
    c = lax.axis_index("c")
    s = lax.axis_index("s")
    wid = c * NS + s

    zeros16 = jnp.zeros((LANES,), jnp.float32)
    ones16 = jnp.ones((LANES,), jnp.float32)

    def _zero_rows_a():
        def _zr(i, carry):
            for j in range(HD // LANES):
                rows_a[i, pl.ds(j * LANES, LANES)] = zeros16
            return carry

        lax.fori_loop(0, ZCH, _zr, 0)

    _zero_rows_a()

    def _init_row(i, carry):
        zcnt_v[i, :] = zeros16
        return carry

    lax.fori_loop(0, ZCH, _init_row, 0)

    def _init_ones(i, carry):
        ones_v[i, :] = ones16
        return carry

    lax.fori_loop(0, CH, _init_ones, 0)

    def _zero_acc():
        for j in range(NZ):
            pltpu.sync_copy(
                rows_a, acc_sh.at[pl.ds(s * SEG_ROWS + j * ZCH, ZCH)])

    _zero_acc()
    for j in range(NZ):
        pltpu.sync_copy(zcnt_v, cnt_sh.at[pl.ds(s * SEG_ROWS + j * ZCH, ZCH)])
    plsc.subcore_barrier()

    @pl.when(c == 0)
    def _():
        pltpu.sync_copy(e0_hbm.at[1, s], src_v)
        pltpu.sync_copy(e0_hbm.at[0, s], dst_v)

    @pl.when(c == 1)
    def _():
        pltpu.sync_copy(e1_hbm.at[1, s], src_v)
        pltpu.sync_copy(e1_hbm.at[0, s], dst_v)

    bufs = (rows_a, rows_b, rows_c, rows_d)

    def _edge_phase(table_hbm, do_cnt):
        def _gather_src(j):
            return table_hbm.at[src_v.at[j]]

        def _rows(buf):
            return buf.at[pl.ds(0, CH)]

        pltpu.async_copy(_gather_src(0), _rows(rows_a), sem_g)
        pltpu.async_copy(_gather_src(1), _rows(rows_b), sem_g)

        def _quad(i, carry):
            for t in range(4):
                j = 4 * i + t
                buf = bufs[t]
                bufm2 = bufs[(t + 2) % 4]
                pltpu.make_async_copy(
                    _gather_src(j), _rows(buf), sem_g).wait()
                pltpu.async_copy(
                    _rows(buf), acc_sh.at[dst_v.at[j]], sem_s, add=True)
                if do_cnt:
                    pltpu.async_copy(
                        ones_v, cnt_sh.at[dst_v.at[j]], sem_c, add=True)

                @pl.when(j >= 2)
                def _():
                    pltpu.make_async_copy(
                        _rows(bufm2), acc_sh.at[dst_v.at[j - 2]], sem_s).wait()

                @pl.when(j + 2 < NCH)
                def _():
                    pltpu.async_copy(_gather_src(j + 2), _rows(bufm2), sem_g)
            return carry

        lax.fori_loop(0, NCH // 4, _quad, 0)

        pltpu.make_async_copy(
            _rows(bufs[2]), acc_sh.at[dst_v.at[NCH - 2]], sem_s).wait()
        pltpu.make_async_copy(
            _rows(bufs[3]), acc_sh.at[dst_v.at[NCH - 1]], sem_s).wait()

        if do_cnt:
            def _drain(j, carry):
                pltpu.make_async_copy(
                    ones_v, cnt_sh.at[dst_v.at[j]], sem_c).wait()
                return carry

            lax.fori_loop(0, NCH, _drain, 0)

    _edge_phase(featl_hbm, True)

    plsc.subcore_barrier()
    for j in range(NZ):
        base = s * SEG_ROWS + j * ZCH
        pltpu.sync_copy(acc_sh.at[pl.ds(base, ZCH)],
                        acc_out.at[2 * c, pl.ds(base, ZCH)])
        pltpu.sync_copy(cnt_sh.at[pl.ds(base, ZCH)],
                        cnt_out.at[c, pl.ds(base, ZCH)])
    _zero_rows_a()
    _zero_acc()
    plsc.subcore_barrier()

    _edge_phase(featr_hbm, False)

    @pl.when(wid < SELF_TILES)
    def _():
        pltpu.sync_copy(nodes_hbm.at[wid], nidx_v)
        for t in range(NSCH):
            pltpu.async_copy(feat_hbm.at[nidx_v.at[t]], srows_v, sem_g).wait()
            pltpu.sync_copy(
                srows_v,
                self_out.at[pl.ds(wid * SELF_PER_TILE + t * SCH, SCH)])

    plsc.subcore_barrier()
    for j in range(NZ):
        base = s * SEG_ROWS + j * ZCH
        pltpu.sync_copy(acc_sh.at[pl.ds(base, ZCH)],
                        acc_out.at[2 * c + 1, pl.ds(base, ZCH)])


BLK = 2000


def _tdot(x, w):
    return lax.dot_general(x, w, (((1,), (1,)), ((), ())),
                           preferred_element_type=jnp.float32)


def _mlp_body(self_ref, a0l_ref, a0r_ref, a1l_ref, a1r_ref, c0_ref, c1_ref,
              w1_ref, b1_ref, w2_ref, b2_ref, out_ref):
    inv0 = 1.0 / jnp.maximum(c0_ref[:, 0:1], 1.0)
    inv1 = 1.0 / jnp.maximum(c1_ref[:, 0:1], 1.0)
    n0 = jnp.concatenate([a0l_ref[:], a0r_ref[:]], axis=1) * inv0
    n1 = jnp.concatenate([a1l_ref[:], a1r_ref[:]], axis=1) * inv1
    h = _tdot(self_ref[:], w1_ref[:, :D])
    h = h + _tdot(n0, w1_ref[:, D:2 * D])
    h = h + _tdot(n1, w1_ref[:, 2 * D:])
    h = jnp.tanh(h + b1_ref[:])
    out_ref[:] = _tdot(h, w2_ref[:]) + b2_ref[:]


def _mlp(self_feats, a0l, a0r, a1l, a1r, c0, c1, w1, b1, w2, b2):
    row_spec = pl.BlockSpec((BLK, D), lambda i: (i, 0))
    half_spec = pl.BlockSpec((BLK, HD), lambda i: (i, 0))
    cnt_spec = pl.BlockSpec((BLK, LANES), lambda i: (i, 0))

    def full(shape):
        return pl.BlockSpec(shape, lambda *_: (0,) * len(shape))

    return pl.pallas_call(
        _mlp_body,
        grid=(B // BLK,),
        in_specs=[row_spec, half_spec, half_spec, half_spec, half_spec,
                  cnt_spec, cnt_spec,
                  full((D, 3 * D)), full((D,)), full((D, D)), full((D,))],
        out_specs=row_spec,
        out_shape=jax.ShapeDtypeStruct((B, D), jnp.float32),
    )(self_feats, a0l, a0r, a1l, a1r, c0, c1, w1, b1, w2, b2)


def kernel(nodes, edge_index_0, edge_index_1, feat_table, W1, b1, W2, b2):
    nodes_r = nodes.astype(jnp.int32).reshape(SELF_TILES, NSCH, SCH)
    e0 = edge_index_0.astype(jnp.int32).reshape(2, NS, NCH, CH)
    e1 = edge_index_1.astype(jnp.int32).reshape(2, NS, NCH, CH)
    featl = feat_table[:, :HD]
    featr = feat_table[:, HD:]

    self_rows, acc, cnt = _aggregate(
        feat_table, featl, featr, nodes_r, e0, e1)

    return _mlp(self_rows, acc[0], acc[1], acc[2], acc[3],
                cnt[0], cnt[1], W1, b1, W2, b2)


# speedup vs baseline: 9.6887x; 1.1252x over previous
import functools

import jax
import jax.numpy as jnp
from jax import lax
from jax.experimental import pallas as pl
from jax.experimental.pallas import tpu as pltpu
from jax.experimental.pallas import tpu_sc as plsc

B = 10000
D = 128
HD = D // 2
E = 160000
NREL = 2
NC = 2
NS = 16
NW = NC * NS
LANES = 16

CH = 125
EDGES_PER_TILE = E // NS
NCH = EDGES_PER_TILE // CH

B_PAD = 10240
SEG_ROWS = B_PAD // NS
ZCH = 128
NZ = SEG_ROWS // ZCH
SELF_TILES = 25
SELF_PER_TILE = B // SELF_TILES
SCH = 80
NSCH = SELF_PER_TILE // SCH

_mesh = plsc.VectorSubcoreMesh(core_axis_name="c", subcore_axis_name="s")


@functools.partial(
    pl.kernel,
    out_type=(
        jax.ShapeDtypeStruct((B, D), jnp.float32),
        jax.ShapeDtypeStruct((NREL, B_PAD, D), jnp.float32),
        jax.ShapeDtypeStruct((NREL, B_PAD, LANES), jnp.float32),
    ),
    mesh=_mesh,
    compiler_params=pltpu.CompilerParams(use_tc_tiling_on_sc=False),
    scratch_types=[
        pltpu.VMEM((NCH, CH), jnp.int32),
        pltpu.VMEM((NCH, CH), jnp.int32),
        pltpu.VMEM((ZCH, HD), jnp.float32),
        pltpu.VMEM((ZCH, HD), jnp.float32),
        pltpu.VMEM((ZCH, HD), jnp.float32),
        pltpu.VMEM((ZCH, HD), jnp.float32),
        pltpu.VMEM((CH, LANES), jnp.float32),
        pltpu.VMEM((NSCH, SCH), jnp.int32),
        pltpu.VMEM((SCH, D), jnp.float32),
        pltpu.VMEM((ZCH, LANES), jnp.float32),
        pltpu.VMEM_SHARED((B_PAD, HD), jnp.float32),
        pltpu.VMEM_SHARED((B_PAD, LANES), jnp.float32),
        pltpu.SemaphoreType.DMA,
        pltpu.SemaphoreType.DMA,
        pltpu.SemaphoreType.DMA,
    ],
)
def _aggregate(feat_hbm, featl_hbm, featr_hbm, nodes_hbm, e0_hbm, e1_hbm,
               self_out, acc_out, cnt_out,
               src_v, dst_v, rows_a, rows_b, rows_c, rows_d,
               ones_v, nidx_v, srows_v,
               zcnt_v, acc_sh, cnt_sh, sem_g, sem_s, sem_c):
    c = lax.axis_index("c")
    s = lax.axis_index("s")
    wid = c * NS + s

    zeros16 = jnp.zeros((LANES,), jnp.float32)
    ones16 = jnp.ones((LANES,), jnp.float32)

    def _zero_rows_a():
        def _zr(i, carry):
            for j in range(HD // LANES):
                rows_a[i, pl.ds(j * LANES, LANES)] = zeros16
            return carry

        lax.fori_loop(0, ZCH, _zr, 0)

    _zero_rows_a()

    def _init_row(i, carry):
        zcnt_v[i, :] = zeros16
        return carry

    lax.fori_loop(0, ZCH, _init_row, 0)

    def _init_ones(i, carry):
        ones_v[i, :] = ones16
        return carry

    lax.fori_loop(0, CH, _init_ones, 0)

    def _zero_acc():
        for j in range(NZ):
            pltpu.sync_copy(
                rows_a, acc_sh.at[pl.ds(s * SEG_ROWS + j * ZCH, ZCH)])

    _zero_acc()
    for j in range(NZ):
        pltpu.sync_copy(zcnt_v, cnt_sh.at[pl.ds(s * SEG_ROWS + j * ZCH, ZCH)])
    plsc.subcore_barrier()

    @pl.when(c == 0)
    def _():
        pltpu.sync_copy(e0_hbm.at[1, s], src_v)
        pltpu.sync_copy(e0_hbm.at[0, s], dst_v)

    @pl.when(c == 1)
    def _():
        pltpu.sync_copy(e1_hbm.at[1, s], src_v)
        pltpu.sync_copy(e1_hbm.at[0, s], dst_v)

    bufs = (rows_a, rows_b, rows_c, rows_d)

    def _edge_phase(table_hbm, do_cnt):
        def _gather_src(j):
            return table_hbm.at[src_v.at[j]]

        def _rows(buf):
            return buf.at[pl.ds(0, CH)]

        pltpu.async_copy(_gather_src(0), _rows(rows_a), sem_g)
        pltpu.async_copy(_gather_src(1), _rows(rows_b), sem_g)

        def _quad(i, carry):
            for t in range(4):
                j = 4 * i + t
                buf = bufs[t]
                bufm2 = bufs[(t + 2) % 4]
                pltpu.make_async_copy(
                    _gather_src(j), _rows(buf), sem_g).wait()
                pltpu.async_copy(
                    _rows(buf), acc_sh.at[dst_v.at[j]], sem_s, add=True)
                if do_cnt:
                    pltpu.async_copy(
                        ones_v, cnt_sh.at[dst_v.at[j]], sem_c, add=True)

                @pl.when(j >= 2)
                def _():
                    pltpu.make_async_copy(
                        _rows(bufm2), acc_sh.at[dst_v.at[j - 2]], sem_s).wait()

                @pl.when(j + 2 < NCH)
                def _():
                    pltpu.async_copy(_gather_src(j + 2), _rows(bufm2), sem_g)
            return carry

        lax.fori_loop(0, NCH // 4, _quad, 0)

        pltpu.make_async_copy(
            _rows(bufs[2]), acc_sh.at[dst_v.at[NCH - 2]], sem_s).wait()
        pltpu.make_async_copy(
            _rows(bufs[3]), acc_sh.at[dst_v.at[NCH - 1]], sem_s).wait()

        if do_cnt:
            def _drain(j, carry):
                pltpu.make_async_copy(
                    ones_v, cnt_sh.at[dst_v.at[j]], sem_c).wait()
                return carry

            lax.fori_loop(0, NCH, _drain, 0)

    _edge_phase(featl_hbm, True)

    plsc.subcore_barrier()
    for j in range(NZ):
        base = s * SEG_ROWS + j * ZCH
        pltpu.sync_copy(acc_sh.at[pl.ds(base, ZCH)],
                        acc_out.at[c, pl.ds(base, ZCH), pl.ds(0, HD)])
        pltpu.sync_copy(cnt_sh.at[pl.ds(base, ZCH)],
                        cnt_out.at[c, pl.ds(base, ZCH)])
    _zero_rows_a()
    _zero_acc()
    plsc.subcore_barrier()

    _edge_phase(featr_hbm, False)

    @pl.when(wid < SELF_TILES)
    def _():
        pltpu.sync_copy(nodes_hbm.at[wid], nidx_v)
        for t in range(NSCH):
            pltpu.async_copy(feat_hbm.at[nidx_v.at[t]], srows_v, sem_g).wait()
            pltpu.sync_copy(
                srows_v,
                self_out.at[pl.ds(wid * SELF_PER_TILE + t * SCH, SCH)])

    plsc.subcore_barrier()
    for j in range(NZ):
        base = s * SEG_ROWS + j * ZCH
        pltpu.sync_copy(acc_sh.at[pl.ds(base, ZCH)],
                        acc_out.at[c, pl.ds(base, ZCH), pl.ds(HD, HD)])


BLK = 2000


def _tdot(x, w):
    return lax.dot_general(x, w, (((1,), (1,)), ((), ())),
                           preferred_element_type=jnp.float32)


def _mlp_body(self_ref, a0_ref, a1_ref, c0_ref, c1_ref,
              w1_ref, b1_ref, w2_ref, b2_ref, out_ref):
    inv0 = 1.0 / jnp.maximum(c0_ref[:, 0:1], 1.0)
    inv1 = 1.0 / jnp.maximum(c1_ref[:, 0:1], 1.0)
    h = _tdot(self_ref[:], w1_ref[:, :D])
    h = h + _tdot(a0_ref[:] * inv0, w1_ref[:, D:2 * D])
    h = h + _tdot(a1_ref[:] * inv1, w1_ref[:, 2 * D:])
    h = jnp.tanh(h + b1_ref[:])
    out_ref[:] = _tdot(h, w2_ref[:]) + b2_ref[:]


def _mlp(self_feats, a0, a1, c0, c1, w1, b1, w2, b2):
    row_spec = pl.BlockSpec((BLK, D), lambda i: (i, 0))
    cnt_spec = pl.BlockSpec((BLK, LANES), lambda i: (i, 0))

    def full(shape):
        return pl.BlockSpec(shape, lambda *_: (0,) * len(shape))

    return pl.pallas_call(
        _mlp_body,
        grid=(B // BLK,),
        in_specs=[row_spec, row_spec, row_spec, cnt_spec, cnt_spec,
                  full((D, 3 * D)), full((D,)), full((D, D)), full((D,))],
        out_specs=row_spec,
        out_shape=jax.ShapeDtypeStruct((B, D), jnp.float32),
    )(self_feats, a0, a1, c0, c1, w1, b1, w2, b2)


def kernel(nodes, edge_index_0, edge_index_1, feat_table, W1, b1, W2, b2):
    nodes_r = nodes.astype(jnp.int32).reshape(SELF_TILES, NSCH, SCH)
    e0 = edge_index_0.astype(jnp.int32).reshape(2, NS, NCH, CH)
    e1 = edge_index_1.astype(jnp.int32).reshape(2, NS, NCH, CH)
    featl = feat_table[:, :HD]
    featr = feat_table[:, HD:]

    self_rows, acc, cnt = _aggregate(
        feat_table, featl, featr, nodes_r, e0, e1)

    return _mlp(self_rows, acc[0], acc[1], cnt[0], cnt[1], W1, b1, W2, b2)
